# baseline (speedup 1.0000x reference)
import functools

import jax
import jax.numpy as jnp
from jax import lax
from jax.experimental import pallas as pl
from jax.experimental.pallas import tpu as pltpu
from jax.experimental.pallas import tpu_sc as plsc

N = 10000
E = 160000
D_IN = 512
D_H = 256
D_OUT = 128

NT = 16

W = 112
EPT = E // NT
NWIN = 90
EPTP = NWIN * W
NPAD = 10112
RPT = NPAD // NT
RPT_FULL = (RPT // W) * W
RPT_REM = RPT - RPT_FULL

EPT2 = E // (2 * NT)
NWIN2 = 48
EPTP2 = NWIN2 * W

DW = 128
DNWIN = 80
DEPTP = DNWIN * DW
DNPAD = 10240
DRPT = DNPAD // NT

_MESH = plsc.VectorSubcoreMesh(core_axis_name="c", subcore_axis_name="s")


def _zero_fill(buf, rows, cols):
    zeros16 = jnp.zeros((16,), jnp.float32)

    @pl.loop(0, rows)
    def _(i):
        @pl.loop(0, cols, step=16)
        def _(j):
            buf[i, pl.ds(j, 16)] = zeros16


@functools.partial(
    pl.kernel,
    out_type=jax.ShapeDtypeStruct((2, DNPAD), jnp.float32),
    mesh=_MESH,
    scratch_types=[
        pltpu.VMEM((DNWIN, DW), jnp.int32),
        pltpu.VMEM((DW,), jnp.float32),
        pltpu.VMEM((DRPT,), jnp.float32),
        pltpu.VMEM_SHARED((DNPAD,), jnp.float32),
    ],
)
def _deg_kernel(idx_hbm, out_hbm, idx_v, ones_v, zer_v, acc):
    c = lax.axis_index("c")
    s = lax.axis_index("s")
    pltpu.sync_copy(idx_hbm.at[c, s], idx_v)

    ones16 = jnp.ones((16,), jnp.float32)
    zeros16 = jnp.zeros((16,), jnp.float32)

    @pl.loop(0, DW, step=16)
    def _(j):
        ones_v[pl.ds(j, 16)] = ones16

    @pl.loop(0, DRPT, step=16)
    def _(j):
        zer_v[pl.ds(j, 16)] = zeros16

    pltpu.sync_copy(zer_v, acc.at[pl.ds(s * DRPT, DRPT)])
    plsc.subcore_barrier()

    @pl.loop(0, DNWIN)
    def _(w):
        pltpu.sync_copy(ones_v, acc.at[idx_v.at[w]], add=True)

    plsc.subcore_barrier()
    pltpu.sync_copy(acc.at[pl.ds(s * DRPT, DRPT)],
                    out_hbm.at[c, pl.ds(s * DRPT, DRPT)])


def _make_agg_kernel(nwin: int, dst_per_core: bool):
    niter = nwin // 6

    @functools.partial(
        pl.kernel,
        out_type=jax.ShapeDtypeStruct((2, NPAD, 128), jnp.float32),
        mesh=_MESH,
        scratch_types=[
            pltpu.VMEM((2, 3, W), jnp.int32),
            pltpu.VMEM((2, 3, W), jnp.int32),
            pltpu.VMEM((3, W, 128), jnp.float32),
            pltpu.VMEM_SHARED((NPAD, 128), jnp.float32),
            pltpu.SemaphoreType.DMA,
            pltpu.SemaphoreType.DMA,
            pltpu.SemaphoreType.DMA,
            pltpu.SemaphoreType.DMA,
            pltpu.SemaphoreType.DMA,
            pltpu.SemaphoreType.DMA,
            pltpu.SemaphoreType.DMA,
            pltpu.SemaphoreType.DMA,
            pltpu.SemaphoreType.DMA,
            pltpu.SemaphoreType.DMA,
        ],
    )
    def _agg(h_hbm, src_hbm, dst_hbm, out_hbm, src_v, dst_v, buf, acc,
             g0, g1, g2, s0, s1, s2, fs0, fs1, fd0, fd1):
        c = lax.axis_index("c")
        s = lax.axis_index("s")
        src_base = src_hbm.at[c, s]
        dst_base = dst_hbm.at[c, s] if dst_per_core else dst_hbm.at[s]
        gsem = (g0, g1, g2)
        ssem = (s0, s1, s2)
        fsrc = (fs0, fs1)
        fdst = (fd0, fd1)

        _zero_fill(buf.at[0], W, 128)

        @pl.loop(0, RPT_FULL, step=W)
        def _(r):
            pltpu.sync_copy(buf.at[0], acc.at[pl.ds(s * RPT + r, W)])

        pltpu.sync_copy(buf.at[0].at[pl.ds(0, RPT_REM)],
                        acc.at[pl.ds(s * RPT + RPT_FULL, RPT_REM)])

        plsc.subcore_barrier()

        def fetch_src(grp, h):
            pltpu.async_copy(src_base.at[grp], src_v.at[h], fsrc[h])

        def wait_fsrc(h):
            pltpu.make_async_copy(src_base.at[0], src_v.at[h],
                                  fsrc[h]).wait()

        def fetch_dst(grp, h):
            pltpu.async_copy(dst_base.at[grp], dst_v.at[h], fdst[h])

        def wait_fdst(h):
            pltpu.make_async_copy(dst_base.at[0], dst_v.at[h],
                                  fdst[h]).wait()

        def gather(h, r, b):
            pltpu.async_copy(h_hbm.at[src_v.at[h].at[r]], buf.at[b],
                             gsem[b])

        def wait_g(b):
            pltpu.make_async_copy(h_hbm.at[pl.ds(0, W)], buf.at[b],
                                  gsem[b]).wait()

        def scatter(b, h, r):
            pltpu.async_copy(buf.at[b], acc.at[dst_v.at[h].at[r]],
                             ssem[b], add=True)

        def wait_s(b):
            pltpu.make_async_copy(buf.at[b], acc.at[pl.ds(0, W)],
                                  ssem[b]).wait()

        fetch_src(0, 0)
        fetch_src(1, 1)
        fetch_dst(0, 0)
        fetch_dst(1, 1)
        wait_fsrc(0)
        gather(0, 0, 0)

        @pl.loop(0, niter)
        def _(m):
            not_first = m > 0
            not_last = m < niter - 1

            wait_fdst(0)
            wait_g(0)
            scatter(0, 0, 0)

            @pl.when(not_first)
            def _():
                wait_s(1)
            gather(0, 1, 1)

            wait_g(1)
            scatter(1, 0, 1)

            @pl.when(not_first)
            def _():
                wait_s(2)
                fetch_dst(2 * m + 1, 1)
            gather(0, 2, 2)

            wait_g(2)
            scatter(2, 0, 2)
            wait_s(0)
            wait_fsrc(1)
            gather(1, 0, 0)

            @pl.when(not_last)
            def _():
                fetch_src(2 * m + 2, 0)

            wait_fdst(1)
            wait_g(0)
            scatter(0, 1, 0)
            wait_s(1)
            gather(1, 1, 1)

            wait_g(1)
            scatter(1, 1, 1)
            wait_s(2)
            gather(1, 2, 2)

            @pl.when(not_last)
            def _():
                fetch_dst(2 * m + 2, 0)

            wait_g(2)
            scatter(2, 1, 2)

            @pl.when(not_last)
            def _():
                wait_s(0)
                wait_fsrc(0)
                gather(0, 0, 0)
                fetch_src(2 * m + 3, 1)

        wait_s(0)
        wait_s(1)
        wait_s(2)

        plsc.subcore_barrier()
        pltpu.sync_copy(acc.at[pl.ds(s * RPT, RPT)],
                        out_hbm.at[c, pl.ds(s * RPT, RPT)])

    return _agg


_agg_l1 = _make_agg_kernel(NWIN, False)
_agg_l2 = _make_agg_kernel(NWIN2, True)


BR = 400
NB = N // BR


def _norm(deg_row):
    return lax.rsqrt(jnp.where(deg_row > 0.0, deg_row, 1.0))


def _mm1_body(x_ref, w_ref, o_ref):
    o_ref[...] = lax.dot_general(
        x_ref[...], w_ref[...], (((1,), (0,)), ((), ())),
        preferred_element_type=jnp.float32,
        precision=lax.Precision.HIGHEST)


def _mm1(x, w1):
    return pl.pallas_call(
        _mm1_body,
        grid=(NB,),
        in_specs=[pl.BlockSpec((BR, D_IN), lambda i: (i, 0)),
                  pl.BlockSpec((D_IN, D_H), lambda i: (0, 0))],
        out_specs=pl.BlockSpec((BR, D_H), lambda i: (i, 0)),
        out_shape=jax.ShapeDtypeStruct((N, D_H), jnp.float32),
    )(x, w1)


def _scale_body(h_ref, deg_ref, o_ref):
    ns = _norm(deg_ref[0][:, 0:1])
    o_ref[0] = h_ref[:, : D_H // 2] * ns
    o_ref[1] = h_ref[:, D_H // 2:] * ns


def _scale(h1, degc):
    return pl.pallas_call(
        _scale_body,
        grid=(NB,),
        in_specs=[pl.BlockSpec((BR, D_H), lambda i: (i, 0)),
                  pl.BlockSpec((1, BR, 2), lambda i: (i, 0, 0))],
        out_specs=pl.BlockSpec((2, BR, D_H // 2), lambda i: (0, i, 0)),
        out_shape=jax.ShapeDtypeStruct((2, N, D_H // 2), jnp.float32),
    )(h1, degc)


def _mid_body(agg_ref, deg_ref, b1_ref, w2_ref, o_ref):
    a = jnp.concatenate([agg_ref[0], agg_ref[1]], axis=-1)
    ns = _norm(deg_ref[0][:, 0:1])
    nd = _norm(deg_ref[0][:, 1:2])
    z = jnp.maximum(a * nd + b1_ref[...], 0.0) * ns
    y = lax.dot_general(z, w2_ref[...], (((1,), (0,)), ((), ())),
                        preferred_element_type=jnp.float32,
                        precision=lax.Precision.HIGHEST)
    o_ref[...] = y


def _mid(agg1, degc, b1, w2):
    return pl.pallas_call(
        _mid_body,
        grid=(NB,),
        in_specs=[pl.BlockSpec((2, BR, D_H // 2), lambda i: (0, i, 0)),
                  pl.BlockSpec((1, BR, 2), lambda i: (i, 0, 0)),
                  pl.BlockSpec((1, D_H), lambda i: (0, 0)),
                  pl.BlockSpec((D_H, D_OUT), lambda i: (0, 0))],
        out_specs=pl.BlockSpec((BR, D_OUT), lambda i: (i, 0)),
        out_shape=jax.ShapeDtypeStruct((N, D_OUT), jnp.float32),
    )(agg1, degc, b1.reshape(1, D_H), w2)


def _fin_body(agg_ref, deg_ref, b2_ref, o_ref):
    a = agg_ref[0] + agg_ref[1]
    nd = _norm(deg_ref[0][:, 1:2])
    o_ref[...] = a * nd + b2_ref[...]


def _fin(agg2, degc, b2):
    return pl.pallas_call(
        _fin_body,
        grid=(NB,),
        in_specs=[pl.BlockSpec((2, BR, D_OUT), lambda i: (0, i, 0)),
                  pl.BlockSpec((1, BR, 2), lambda i: (i, 0, 0)),
                  pl.BlockSpec((1, D_OUT), lambda i: (0, 0))],
        out_specs=pl.BlockSpec((BR, D_OUT), lambda i: (i, 0)),
        out_shape=jax.ShapeDtypeStruct((N, D_OUT), jnp.float32),
    )(agg2, degc, b2.reshape(1, D_OUT))


def kernel(x, edge_index, W1, b1, W2, b2):
    src = edge_index[0].astype(jnp.int32).reshape(NT, EPT)
    dst = edge_index[1].astype(jnp.int32).reshape(NT, EPT)

    npad1 = EPTP - EPT
    pad_read = (jnp.arange(npad1, dtype=jnp.int32) * 41) % N
    pad_hi = N + (jnp.arange(npad1, dtype=jnp.int32) % (NPAD - N))
    pad_deg = N + (jnp.arange(DEPTP - EPT, dtype=jnp.int32) % (DNPAD - N))

    def _slab(idx, pad, nwin, w):
        npd = nwin * w - idx.shape[1]
        return jnp.concatenate(
            [idx, jnp.broadcast_to(pad[:npd], (NT, npd))], axis=1
        ).reshape(NT, nwin, w)

    src_gather = _slab(src, pad_read, NWIN, W)
    src_slab = jnp.stack([src_gather, src_gather + N]
                         ).reshape(2, NT, NWIN // 3, 3, W)
    dst_slab = _slab(dst, pad_hi, NWIN, W).reshape(NT, NWIN // 3, 3, W)
    deg_slab = jnp.stack([_slab(src, pad_deg, DNWIN, DW),
                          _slab(dst, pad_deg, DNWIN, DW)])

    npad2 = EPTP2 - EPT2
    pad_read2 = (jnp.arange(npad2, dtype=jnp.int32) * 41) % N
    pad_hi2 = N + (jnp.arange(npad2, dtype=jnp.int32) % (NPAD - N))

    def _slab2(idx, pad):
        return jnp.concatenate(
            [idx.reshape(2, NT, EPT2),
             jnp.broadcast_to(pad, (2, NT, npad2))], axis=2
        ).reshape(2, NT, NWIN2, W)

    src2_slab = _slab2(src, pad_read2).reshape(2, NT, NWIN2 // 3, 3, W)
    dst2_slab = _slab2(dst, pad_hi2).reshape(2, NT, NWIN2 // 3, 3, W)

    deg = _deg_kernel(deg_slab)
    degc = deg[:, :N].T.reshape(NB, BR, 2)
    h1 = _mm1(x, W1)
    hcat1 = _scale(h1, degc).reshape(2 * N, D_H // 2)
    agg1 = _agg_l1(hcat1, src_slab, dst_slab)
    h2 = _mid(agg1, degc, b1, W2)
    agg2 = _agg_l2(h2, src2_slab, dst2_slab)
    return _fin(agg2, degc, b2)


# speedup vs baseline: 1.1266x; 1.1266x over previous
import functools

import jax
import jax.numpy as jnp
from jax import lax
from jax.experimental import pallas as pl
from jax.experimental.pallas import tpu as pltpu
from jax.experimental.pallas import tpu_sc as plsc

N = 10000
E = 160000
D_IN = 512
D_H = 256
D_OUT = 128

NT = 16

W = 112
EPT = E // NT
NWIN = 92
EPTP = NWIN * W
NPAD = 10112
RPT = NPAD // NT
RPT_FULL = (RPT // W) * W
RPT_REM = RPT - RPT_FULL

EPT2 = E // (2 * NT)
NWIN2 = 48
EPTP2 = NWIN2 * W

DW = 128
DNWIN = 80
DEPTP = DNWIN * DW
DNPAD = 10240
DRPT = DNPAD // NT

_MESH = plsc.VectorSubcoreMesh(core_axis_name="c", subcore_axis_name="s")


def _zero_fill(buf, rows, cols):
    zeros16 = jnp.zeros((16,), jnp.float32)

    @pl.loop(0, rows)
    def _(i):
        @pl.loop(0, cols, step=16)
        def _(j):
            buf[i, pl.ds(j, 16)] = zeros16


@functools.partial(
    pl.kernel,
    out_type=jax.ShapeDtypeStruct((2, DNPAD), jnp.float32),
    mesh=_MESH,
    scratch_types=[
        pltpu.VMEM((DNWIN, DW), jnp.int32),
        pltpu.VMEM((DW,), jnp.float32),
        pltpu.VMEM((DRPT,), jnp.float32),
        pltpu.VMEM_SHARED((DNPAD,), jnp.float32),
    ],
)
def _deg_kernel(idx_hbm, out_hbm, idx_v, ones_v, zer_v, acc):
    c = lax.axis_index("c")
    s = lax.axis_index("s")
    pltpu.sync_copy(idx_hbm.at[c, s], idx_v)

    ones16 = jnp.ones((16,), jnp.float32)
    zeros16 = jnp.zeros((16,), jnp.float32)

    @pl.loop(0, DW, step=16)
    def _(j):
        ones_v[pl.ds(j, 16)] = ones16

    @pl.loop(0, DRPT, step=16)
    def _(j):
        zer_v[pl.ds(j, 16)] = zeros16

    pltpu.sync_copy(zer_v, acc.at[pl.ds(s * DRPT, DRPT)])
    plsc.subcore_barrier()

    @pl.loop(0, DNWIN)
    def _(w):
        pltpu.sync_copy(ones_v, acc.at[idx_v.at[w]], add=True)

    plsc.subcore_barrier()
    pltpu.sync_copy(acc.at[pl.ds(s * DRPT, DRPT)],
                    out_hbm.at[c, pl.ds(s * DRPT, DRPT)])


def _make_agg_kernel(nwin: int, dst_per_core: bool):
    @functools.partial(
        pl.kernel,
        out_type=jax.ShapeDtypeStruct((2, NPAD, 128), jnp.float32),
        mesh=_MESH,
        scratch_types=[
            pltpu.VMEM((nwin, W), jnp.int32),
            pltpu.VMEM((2, 2, W), jnp.int32),
            pltpu.VMEM((2, W, 128), jnp.float32),
            pltpu.VMEM_SHARED((NPAD, 128), jnp.float32),
            pltpu.SemaphoreType.DMA,
            pltpu.SemaphoreType.DMA,
            pltpu.SemaphoreType.DMA,
            pltpu.SemaphoreType.DMA,
        ],
    )
    def _agg(h_hbm, src_hbm, dst_hbm, out_hbm, src_v, dst_v, buf, acc,
             sem0, sem1, dsem0, dsem1):
        c = lax.axis_index("c")
        s = lax.axis_index("s")
        pltpu.sync_copy(src_hbm.at[c, s], src_v)
        dst_base = dst_hbm.at[c, s] if dst_per_core else dst_hbm.at[s]

        _zero_fill(buf.at[0], W, 128)

        @pl.loop(0, RPT_FULL, step=W)
        def _(r):
            pltpu.sync_copy(buf.at[0], acc.at[pl.ds(s * RPT + r, W)])

        pltpu.sync_copy(buf.at[0].at[pl.ds(0, RPT_REM)],
                        acc.at[pl.ds(s * RPT + RPT_FULL, RPT_REM)])

        plsc.subcore_barrier()

        dsems = (dsem0, dsem1)

        def _wait_g(k, sem):
            pltpu.make_async_copy(h_hbm.at[pl.ds(0, W)], buf.at[k], sem).wait()

        def _gather(w, k, sem):
            pltpu.async_copy(h_hbm.at[src_v.at[w]], buf.at[k], sem)

        def _dst_fetch(pair, k):
            pltpu.async_copy(dst_base.at[pair], dst_v.at[k], dsems[k])

        def _wait_dst(k):
            pltpu.make_async_copy(dst_base.at[0], dst_v.at[k],
                                  dsems[k]).wait()

        def _scatter(bufk, dstk, row):
            pltpu.sync_copy(buf.at[bufk], acc.at[dst_v.at[dstk].at[row]],
                            add=True)

        _dst_fetch(0, 0)
        _dst_fetch(1, 1)
        _gather(0, 0, sem0)

        @pl.loop(0, nwin, step=4)
        def _(w):
            _gather(w + 1, 1, sem1)
            _wait_dst(0)
            _wait_g(0, sem0)
            _scatter(0, 0, 0)

            @pl.when(w + 2 < nwin)
            def _():
                _gather(w + 2, 0, sem0)

            _wait_g(1, sem1)
            _scatter(1, 0, 1)

            @pl.when(w + 4 < nwin)
            def _():
                _dst_fetch(w // 2 + 2, 0)
                _gather(w + 3, 1, sem1)

            @pl.when(w + 4 >= nwin)
            def _():
                _gather(w + 3, 1, sem1)

            _wait_dst(1)
            _wait_g(0, sem0)
            _scatter(0, 1, 0)

            @pl.when(w + 4 < nwin)
            def _():
                _gather(w + 4, 0, sem0)

            _wait_g(1, sem1)
            _scatter(1, 1, 1)

            @pl.when(w + 6 < nwin)
            def _():
                _dst_fetch(w // 2 + 3, 1)

        plsc.subcore_barrier()
        pltpu.sync_copy(acc.at[pl.ds(s * RPT, RPT)],
                        out_hbm.at[c, pl.ds(s * RPT, RPT)])

    return _agg


_agg_l1 = _make_agg_kernel(NWIN, False)
_agg_l2 = _make_agg_kernel(NWIN2, True)


BR = 400
NB = N // BR


def _norm(deg_row):
    return lax.rsqrt(jnp.where(deg_row > 0.0, deg_row, 1.0))


def _mm1_body(x_ref, w_ref, o_ref):
    o_ref[...] = lax.dot_general(
        x_ref[...], w_ref[...], (((1,), (0,)), ((), ())),
        preferred_element_type=jnp.float32,
        precision=lax.Precision.HIGHEST)


def _mm1(x, w1):
    return pl.pallas_call(
        _mm1_body,
        grid=(NB,),
        in_specs=[pl.BlockSpec((BR, D_IN), lambda i: (i, 0)),
                  pl.BlockSpec((D_IN, D_H), lambda i: (0, 0))],
        out_specs=pl.BlockSpec((BR, D_H), lambda i: (i, 0)),
        out_shape=jax.ShapeDtypeStruct((N, D_H), jnp.float32),
    )(x, w1)


def _scale_body(h_ref, deg_ref, o_ref):
    ns = _norm(deg_ref[0][:, 0:1])
    o_ref[0] = h_ref[:, : D_H // 2] * ns
    o_ref[1] = h_ref[:, D_H // 2:] * ns


def _scale(h1, degc):
    return pl.pallas_call(
        _scale_body,
        grid=(NB,),
        in_specs=[pl.BlockSpec((BR, D_H), lambda i: (i, 0)),
                  pl.BlockSpec((1, BR, 2), lambda i: (i, 0, 0))],
        out_specs=pl.BlockSpec((2, BR, D_H // 2), lambda i: (0, i, 0)),
        out_shape=jax.ShapeDtypeStruct((2, N, D_H // 2), jnp.float32),
    )(h1, degc)


def _mid_body(agg_ref, deg_ref, b1_ref, w2_ref, o_ref):
    a = jnp.concatenate([agg_ref[0], agg_ref[1]], axis=-1)
    ns = _norm(deg_ref[0][:, 0:1])
    nd = _norm(deg_ref[0][:, 1:2])
    z = jnp.maximum(a * nd + b1_ref[...], 0.0) * ns
    y = lax.dot_general(z, w2_ref[...], (((1,), (0,)), ((), ())),
                        preferred_element_type=jnp.float32,
                        precision=lax.Precision.HIGHEST)
    o_ref[...] = y


def _mid(agg1, degc, b1, w2):
    return pl.pallas_call(
        _mid_body,
        grid=(NB,),
        in_specs=[pl.BlockSpec((2, BR, D_H // 2), lambda i: (0, i, 0)),
                  pl.BlockSpec((1, BR, 2), lambda i: (i, 0, 0)),
                  pl.BlockSpec((1, D_H), lambda i: (0, 0)),
                  pl.BlockSpec((D_H, D_OUT), lambda i: (0, 0))],
        out_specs=pl.BlockSpec((BR, D_OUT), lambda i: (i, 0)),
        out_shape=jax.ShapeDtypeStruct((N, D_OUT), jnp.float32),
    )(agg1, degc, b1.reshape(1, D_H), w2)


def _fin_body(agg_ref, deg_ref, b2_ref, o_ref):
    a = agg_ref[0] + agg_ref[1]
    nd = _norm(deg_ref[0][:, 1:2])
    o_ref[...] = a * nd + b2_ref[...]


def _fin(agg2, degc, b2):
    return pl.pallas_call(
        _fin_body,
        grid=(NB,),
        in_specs=[pl.BlockSpec((2, BR, D_OUT), lambda i: (0, i, 0)),
                  pl.BlockSpec((1, BR, 2), lambda i: (i, 0, 0)),
                  pl.BlockSpec((1, D_OUT), lambda i: (0, 0))],
        out_specs=pl.BlockSpec((BR, D_OUT), lambda i: (i, 0)),
        out_shape=jax.ShapeDtypeStruct((N, D_OUT), jnp.float32),
    )(agg2, degc, b2.reshape(1, D_OUT))


def kernel(x, edge_index, W1, b1, W2, b2):
    src = edge_index[0].astype(jnp.int32).reshape(NT, EPT)
    dst = edge_index[1].astype(jnp.int32).reshape(NT, EPT)

    npad1 = EPTP - EPT
    pad_read = (jnp.arange(npad1, dtype=jnp.int32) * 41) % N
    pad_hi = N + (jnp.arange(npad1, dtype=jnp.int32) % (NPAD - N))
    pad_deg = N + (jnp.arange(DEPTP - EPT, dtype=jnp.int32) % (DNPAD - N))

    def _slab(idx, pad, nwin, w):
        npd = nwin * w - idx.shape[1]
        return jnp.concatenate(
            [idx, jnp.broadcast_to(pad[:npd], (NT, npd))], axis=1
        ).reshape(NT, nwin, w)

    src_gather = _slab(src, pad_read, NWIN, W)
    src_slab = jnp.stack([src_gather, src_gather + N])
    dst_slab = _slab(dst, pad_hi, NWIN, W).reshape(NT, NWIN // 2, 2, W)
    deg_slab = jnp.stack([_slab(src, pad_deg, DNWIN, DW),
                          _slab(dst, pad_deg, DNWIN, DW)])

    npad2 = EPTP2 - EPT2
    pad_read2 = (jnp.arange(npad2, dtype=jnp.int32) * 41) % N
    pad_hi2 = N + (jnp.arange(npad2, dtype=jnp.int32) % (NPAD - N))

    def _slab2(idx, pad):
        return jnp.concatenate(
            [idx.reshape(2, NT, EPT2),
             jnp.broadcast_to(pad, (2, NT, npad2))], axis=2
        ).reshape(2, NT, NWIN2, W)

    src2_slab = _slab2(src, pad_read2)
    dst2_slab = _slab2(dst, pad_hi2).reshape(2, NT, NWIN2 // 2, 2, W)

    deg = _deg_kernel(deg_slab)
    degc = deg[:, :N].T.reshape(NB, BR, 2)
    h1 = _mm1(x, W1)
    hcat1 = _scale(h1, degc).reshape(2 * N, D_H // 2)
    agg1 = _agg_l1(hcat1, src_slab, dst_slab)
    h2 = _mid(agg1, degc, b1, W2)
    agg2 = _agg_l2(h2, src2_slab, dst2_slab)
    return _fin(agg2, degc, b2)


# speedup vs baseline: 1.1570x; 1.0270x over previous
import functools

import jax
import jax.numpy as jnp
from jax import lax
from jax.experimental import pallas as pl
from jax.experimental.pallas import tpu as pltpu
from jax.experimental.pallas import tpu_sc as plsc

N = 10000
E = 160000
D_IN = 512
D_H = 256
D_OUT = 128

NT = 16

W = 128
EPT = E // NT
NWIN = 80
EPTP = NWIN * W
NPAD = 10112
RPT = NPAD // NT
RPT_FULL = (RPT // W) * W
RPT_REM = RPT - RPT_FULL

EPT2 = E // (2 * NT)
NWIN2 = 40
EPTP2 = NWIN2 * W

DW = 128
DNWIN = 80
DEPTP = DNWIN * DW
DNPAD = 10240
DRPT = DNPAD // NT

_MESH = plsc.VectorSubcoreMesh(core_axis_name="c", subcore_axis_name="s")


def _zero_fill(buf, rows, cols):
    zeros16 = jnp.zeros((16,), jnp.float32)

    @pl.loop(0, rows)
    def _(i):
        @pl.loop(0, cols, step=16)
        def _(j):
            buf[i, pl.ds(j, 16)] = zeros16


@functools.partial(
    pl.kernel,
    out_type=jax.ShapeDtypeStruct((2, DNPAD), jnp.float32),
    mesh=_MESH,
    scratch_types=[
        pltpu.VMEM((DNWIN, DW), jnp.int32),
        pltpu.VMEM((DW,), jnp.float32),
        pltpu.VMEM((DRPT,), jnp.float32),
        pltpu.VMEM_SHARED((DNPAD,), jnp.float32),
    ],
)
def _deg_kernel(idx_hbm, out_hbm, idx_v, ones_v, zer_v, acc):
    c = lax.axis_index("c")
    s = lax.axis_index("s")
    pltpu.sync_copy(idx_hbm.at[c, s], idx_v)

    ones16 = jnp.ones((16,), jnp.float32)
    zeros16 = jnp.zeros((16,), jnp.float32)

    @pl.loop(0, DW, step=16)
    def _(j):
        ones_v[pl.ds(j, 16)] = ones16

    @pl.loop(0, DRPT, step=16)
    def _(j):
        zer_v[pl.ds(j, 16)] = zeros16

    pltpu.sync_copy(zer_v, acc.at[pl.ds(s * DRPT, DRPT)])
    plsc.subcore_barrier()

    @pl.loop(0, DNWIN)
    def _(w):
        pltpu.sync_copy(ones_v, acc.at[idx_v.at[w]], add=True)

    plsc.subcore_barrier()
    pltpu.sync_copy(acc.at[pl.ds(s * DRPT, DRPT)],
                    out_hbm.at[c, pl.ds(s * DRPT, DRPT)])


def _make_agg_kernel(nwin: int, dst_per_core: bool):
    @functools.partial(
        pl.kernel,
        out_type=jax.ShapeDtypeStruct((2, NPAD, 128), jnp.float32),
        mesh=_MESH,
        scratch_types=[
            pltpu.VMEM((nwin, W), jnp.int32),
            pltpu.VMEM((2, 2, W), jnp.int32),
            pltpu.VMEM((2, W, 128), jnp.float32),
            pltpu.VMEM_SHARED((NPAD, 128), jnp.float32),
            pltpu.SemaphoreType.DMA,
            pltpu.SemaphoreType.DMA,
            pltpu.SemaphoreType.DMA,
            pltpu.SemaphoreType.DMA,
        ],
    )
    def _agg(h_hbm, src_hbm, dst_hbm, out_hbm, src_v, dst_v, buf, acc,
             sem0, sem1, dsem0, dsem1):
        c = lax.axis_index("c")
        s = lax.axis_index("s")
        pltpu.sync_copy(src_hbm.at[c, s], src_v)
        dst_base = dst_hbm.at[c, s] if dst_per_core else dst_hbm.at[s]

        _zero_fill(buf.at[0], W, 128)

        @pl.loop(0, RPT_FULL, step=W)
        def _(r):
            pltpu.sync_copy(buf.at[0], acc.at[pl.ds(s * RPT + r, W)])

        pltpu.sync_copy(buf.at[0].at[pl.ds(0, RPT_REM)],
                        acc.at[pl.ds(s * RPT + RPT_FULL, RPT_REM)])

        plsc.subcore_barrier()

        dsems = (dsem0, dsem1)

        def _wait_g(k, sem):
            pltpu.make_async_copy(h_hbm.at[pl.ds(0, W)], buf.at[k], sem).wait()

        def _gather(w, k, sem):
            pltpu.async_copy(h_hbm.at[src_v.at[w]], buf.at[k], sem)

        def _dst_fetch(pair, k):
            pltpu.async_copy(dst_base.at[pair], dst_v.at[k], dsems[k])

        def _wait_dst(k):
            pltpu.make_async_copy(dst_base.at[0], dst_v.at[k],
                                  dsems[k]).wait()

        def _scatter(bufk, dstk, row):
            pltpu.sync_copy(buf.at[bufk], acc.at[dst_v.at[dstk].at[row]],
                            add=True)

        _dst_fetch(0, 0)
        _dst_fetch(1, 1)
        _gather(0, 0, sem0)

        @pl.loop(0, nwin, step=4)
        def _(w):
            _gather(w + 1, 1, sem1)
            _wait_dst(0)
            _wait_g(0, sem0)
            _scatter(0, 0, 0)

            @pl.when(w + 2 < nwin)
            def _():
                _gather(w + 2, 0, sem0)

            _wait_g(1, sem1)
            _scatter(1, 0, 1)

            @pl.when(w + 4 < nwin)
            def _():
                _dst_fetch(w // 2 + 2, 0)
                _gather(w + 3, 1, sem1)

            @pl.when(w + 4 >= nwin)
            def _():
                _gather(w + 3, 1, sem1)

            _wait_dst(1)
            _wait_g(0, sem0)
            _scatter(0, 1, 0)

            @pl.when(w + 4 < nwin)
            def _():
                _gather(w + 4, 0, sem0)

            _wait_g(1, sem1)
            _scatter(1, 1, 1)

            @pl.when(w + 6 < nwin)
            def _():
                _dst_fetch(w // 2 + 3, 1)

        plsc.subcore_barrier()
        pltpu.sync_copy(acc.at[pl.ds(s * RPT, RPT)],
                        out_hbm.at[c, pl.ds(s * RPT, RPT)])

    return _agg


_agg_l1 = _make_agg_kernel(NWIN, False)
_agg_l2 = _make_agg_kernel(NWIN2, True)


BR = 400
NB = N // BR


def _norm(deg_row):
    return lax.rsqrt(jnp.where(deg_row > 0.0, deg_row, 1.0))


def _mm1_body(x_ref, w_ref, o_ref):
    o_ref[...] = lax.dot_general(
        x_ref[...], w_ref[...], (((1,), (0,)), ((), ())),
        preferred_element_type=jnp.float32,
        precision=lax.Precision.HIGHEST)


def _mm1(x, w1):
    return pl.pallas_call(
        _mm1_body,
        grid=(NB,),
        in_specs=[pl.BlockSpec((BR, D_IN), lambda i: (i, 0)),
                  pl.BlockSpec((D_IN, D_H), lambda i: (0, 0))],
        out_specs=pl.BlockSpec((BR, D_H), lambda i: (i, 0)),
        out_shape=jax.ShapeDtypeStruct((N, D_H), jnp.float32),
    )(x, w1)


def _scale_body(h_ref, deg_ref, o_ref):
    ns = _norm(deg_ref[0][:, 0:1])
    o_ref[0] = h_ref[:, : D_H // 2] * ns
    o_ref[1] = h_ref[:, D_H // 2:] * ns


def _scale(h1, degc):
    return pl.pallas_call(
        _scale_body,
        grid=(NB,),
        in_specs=[pl.BlockSpec((BR, D_H), lambda i: (i, 0)),
                  pl.BlockSpec((1, BR, 2), lambda i: (i, 0, 0))],
        out_specs=pl.BlockSpec((2, BR, D_H // 2), lambda i: (0, i, 0)),
        out_shape=jax.ShapeDtypeStruct((2, N, D_H // 2), jnp.float32),
    )(h1, degc)


def _mid_body(agg_ref, deg_ref, b1_ref, w2_ref, o_ref):
    a = jnp.concatenate([agg_ref[0], agg_ref[1]], axis=-1)
    ns = _norm(deg_ref[0][:, 0:1])
    nd = _norm(deg_ref[0][:, 1:2])
    z = jnp.maximum(a * nd + b1_ref[...], 0.0) * ns
    y = lax.dot_general(z, w2_ref[...], (((1,), (0,)), ((), ())),
                        preferred_element_type=jnp.float32,
                        precision=lax.Precision.HIGHEST)
    o_ref[...] = y


def _mid(agg1, degc, b1, w2):
    return pl.pallas_call(
        _mid_body,
        grid=(NB,),
        in_specs=[pl.BlockSpec((2, BR, D_H // 2), lambda i: (0, i, 0)),
                  pl.BlockSpec((1, BR, 2), lambda i: (i, 0, 0)),
                  pl.BlockSpec((1, D_H), lambda i: (0, 0)),
                  pl.BlockSpec((D_H, D_OUT), lambda i: (0, 0))],
        out_specs=pl.BlockSpec((BR, D_OUT), lambda i: (i, 0)),
        out_shape=jax.ShapeDtypeStruct((N, D_OUT), jnp.float32),
    )(agg1, degc, b1.reshape(1, D_H), w2)


def _fin_body(agg_ref, deg_ref, b2_ref, o_ref):
    a = agg_ref[0] + agg_ref[1]
    nd = _norm(deg_ref[0][:, 1:2])
    o_ref[...] = a * nd + b2_ref[...]


def _fin(agg2, degc, b2):
    return pl.pallas_call(
        _fin_body,
        grid=(NB,),
        in_specs=[pl.BlockSpec((2, BR, D_OUT), lambda i: (0, i, 0)),
                  pl.BlockSpec((1, BR, 2), lambda i: (i, 0, 0)),
                  pl.BlockSpec((1, D_OUT), lambda i: (0, 0))],
        out_specs=pl.BlockSpec((BR, D_OUT), lambda i: (i, 0)),
        out_shape=jax.ShapeDtypeStruct((N, D_OUT), jnp.float32),
    )(agg2, degc, b2.reshape(1, D_OUT))


def kernel(x, edge_index, W1, b1, W2, b2):
    src = edge_index[0].astype(jnp.int32).reshape(NT, EPT)
    dst = edge_index[1].astype(jnp.int32).reshape(NT, EPT)

    npad1 = EPTP - EPT
    pad_read = (jnp.arange(npad1, dtype=jnp.int32) * 41) % N
    pad_hi = N + (jnp.arange(npad1, dtype=jnp.int32) % (NPAD - N))
    pad_deg = N + (jnp.arange(DEPTP - EPT, dtype=jnp.int32) % (DNPAD - N))

    def _slab(idx, pad, nwin, w):
        npd = nwin * w - idx.shape[1]
        return jnp.concatenate(
            [idx, jnp.broadcast_to(pad[:npd], (NT, npd))], axis=1
        ).reshape(NT, nwin, w)

    src_gather = _slab(src, pad_read, NWIN, W)
    src_slab = jnp.stack([src_gather, src_gather + N])
    dst_slab = _slab(dst, pad_hi, NWIN, W).reshape(NT, NWIN // 2, 2, W)
    deg_slab = jnp.stack([_slab(src, pad_deg, DNWIN, DW),
                          _slab(dst, pad_deg, DNWIN, DW)])

    npad2 = EPTP2 - EPT2
    pad_read2 = (jnp.arange(npad2, dtype=jnp.int32) * 41) % N
    pad_hi2 = N + (jnp.arange(npad2, dtype=jnp.int32) % (NPAD - N))

    def _slab2(idx, pad):
        return jnp.concatenate(
            [idx.reshape(2, NT, EPT2),
             jnp.broadcast_to(pad, (2, NT, npad2))], axis=2
        ).reshape(2, NT, NWIN2, W)

    src2_slab = _slab2(src, pad_read2)
    dst2_slab = _slab2(dst, pad_hi2).reshape(2, NT, NWIN2 // 2, 2, W)

    deg = _deg_kernel(deg_slab)
    degc = deg[:, :N].T.reshape(NB, BR, 2)
    h1 = _mm1(x, W1)
    hcat1 = _scale(h1, degc).reshape(2 * N, D_H // 2)
    agg1 = _agg_l1(hcat1, src_slab, dst_slab)
    h2 = _mid(agg1, degc, b1, W2)
    agg2 = _agg_l2(h2, src2_slab, dst2_slab)
    return _fin(agg2, degc, b2)


# speedup vs baseline: 1.2919x; 1.1166x over previous
import functools

import jax
import jax.numpy as jnp
from jax import lax
from jax.experimental import pallas as pl
from jax.experimental.pallas import tpu as pltpu
from jax.experimental.pallas import tpu_sc as plsc

N = 10000
E = 160000
D_IN = 512
D_H = 256
D_OUT = 128

NT = 16

W = 128
EPT = E // NT
NWIN = 80
EPTP = NWIN * W
NPAD = 10112
RPT = NPAD // NT
RPT_FULL = (RPT // W) * W
RPT_REM = RPT - RPT_FULL

EPT2 = E // (2 * NT)
NWIN2 = 40
EPTP2 = NWIN2 * W

DW = 128
DNWIN = 80
DEPTP = DNWIN * DW
DNPAD = 10240
DRPT = DNPAD // NT

_MESH = plsc.VectorSubcoreMesh(core_axis_name="c", subcore_axis_name="s")


def _zero_fill(buf, rows, cols):
    zeros16 = jnp.zeros((16,), jnp.float32)

    @pl.loop(0, rows)
    def _(i):
        @pl.loop(0, cols, step=16)
        def _(j):
            buf[i, pl.ds(j, 16)] = zeros16


@functools.partial(
    pl.kernel,
    out_type=jax.ShapeDtypeStruct((2, DNPAD), jnp.float32),
    mesh=_MESH,
    scratch_types=[
        pltpu.VMEM((DNWIN, DW), jnp.int32),
        pltpu.VMEM((DW,), jnp.float32),
        pltpu.VMEM((DRPT,), jnp.float32),
        pltpu.VMEM_SHARED((DNPAD,), jnp.float32),
    ],
)
def _deg_kernel(idx_hbm, out_hbm, idx_v, ones_v, zer_v, acc):
    c = lax.axis_index("c")
    s = lax.axis_index("s")
    pltpu.sync_copy(idx_hbm.at[c, s], idx_v)

    ones16 = jnp.ones((16,), jnp.float32)
    zeros16 = jnp.zeros((16,), jnp.float32)

    @pl.loop(0, DW, step=16)
    def _(j):
        ones_v[pl.ds(j, 16)] = ones16

    @pl.loop(0, DRPT, step=16)
    def _(j):
        zer_v[pl.ds(j, 16)] = zeros16

    pltpu.sync_copy(zer_v, acc.at[pl.ds(s * DRPT, DRPT)])
    plsc.subcore_barrier()

    @pl.loop(0, DNWIN)
    def _(w):
        pltpu.sync_copy(ones_v, acc.at[idx_v.at[w]], add=True)

    plsc.subcore_barrier()
    pltpu.sync_copy(acc.at[pl.ds(s * DRPT, DRPT)],
                    out_hbm.at[c, pl.ds(s * DRPT, DRPT)])


def _make_agg_kernel(nwin: int, dst_per_core: bool):
    @functools.partial(
        pl.kernel,
        out_type=jax.ShapeDtypeStruct((2, NPAD, 128), jnp.float32),
        mesh=_MESH,
        scratch_types=[
            pltpu.VMEM((nwin, W), jnp.int32),
            pltpu.VMEM((2, 2, W), jnp.int32),
            pltpu.VMEM((2, W, 128), jnp.float32),
            pltpu.VMEM_SHARED((NPAD, 128), jnp.float32),
            pltpu.SemaphoreType.DMA,
            pltpu.SemaphoreType.DMA,
            pltpu.SemaphoreType.DMA,
            pltpu.SemaphoreType.DMA,
        ],
    )
    def _agg(h_hbm, src_hbm, dst_hbm, out_hbm, src_v, dst_v, buf, acc,
             sem0, sem1, dsem0, dsem1):
        c = lax.axis_index("c")
        s = lax.axis_index("s")
        pltpu.sync_copy(src_hbm.at[c, s], src_v)
        dst_base = dst_hbm.at[c, s] if dst_per_core else dst_hbm.at[s]

        _zero_fill(buf.at[0], W, 128)

        @pl.loop(0, RPT_FULL, step=W)
        def _(r):
            pltpu.sync_copy(buf.at[0], acc.at[pl.ds(s * RPT + r, W)])

        pltpu.sync_copy(buf.at[0].at[pl.ds(0, RPT_REM)],
                        acc.at[pl.ds(s * RPT + RPT_FULL, RPT_REM)])

        plsc.subcore_barrier()

        dsems = (dsem0, dsem1)

        def _wait_g(k, sem):
            pltpu.make_async_copy(h_hbm.at[pl.ds(0, W)], buf.at[k], sem).wait()

        def _gather(w, k, sem):
            pltpu.async_copy(h_hbm.at[src_v.at[w]], buf.at[k], sem)

        def _dst_fetch(pair, k):
            pltpu.async_copy(dst_base.at[pair], dst_v.at[k], dsems[k])

        def _wait_dst(k):
            pltpu.make_async_copy(dst_base.at[0], dst_v.at[k],
                                  dsems[k]).wait()

        def _scatter(bufk, dstk, row):
            pltpu.sync_copy(buf.at[bufk], acc.at[dst_v.at[dstk].at[row]],
                            add=True)

        _dst_fetch(0, 0)
        _dst_fetch(1, 1)
        _gather(0, 0, sem0)

        @pl.loop(0, nwin, step=4)
        def _(w):
            _gather(w + 1, 1, sem1)
            _wait_dst(0)
            _wait_g(0, sem0)
            _scatter(0, 0, 0)

            @pl.when(w + 2 < nwin)
            def _():
                _gather(w + 2, 0, sem0)

            _wait_g(1, sem1)
            _scatter(1, 0, 1)

            @pl.when(w + 4 < nwin)
            def _():
                _dst_fetch(w // 2 + 2, 0)
                _gather(w + 3, 1, sem1)

            @pl.when(w + 4 >= nwin)
            def _():
                _gather(w + 3, 1, sem1)

            _wait_dst(1)
            _wait_g(0, sem0)
            _scatter(0, 1, 0)

            @pl.when(w + 4 < nwin)
            def _():
                _gather(w + 4, 0, sem0)

            _wait_g(1, sem1)
            _scatter(1, 1, 1)

            @pl.when(w + 6 < nwin)
            def _():
                _dst_fetch(w // 2 + 3, 1)

        plsc.subcore_barrier()
        pltpu.sync_copy(acc.at[pl.ds(s * RPT, RPT)],
                        out_hbm.at[c, pl.ds(s * RPT, RPT)])

    return _agg


_agg_l1 = _make_agg_kernel(NWIN, False)
_agg_l2 = _make_agg_kernel(NWIN2, True)


BR = 1000
NB = N // BR


def _norm(deg_row):
    return lax.rsqrt(jnp.where(deg_row > 0.0, deg_row, 1.0))


def _mm1_body(x_ref, w_ref, o_ref):
    o_ref[...] = lax.dot_general(
        x_ref[...], w_ref[...], (((1,), (0,)), ((), ())),
        preferred_element_type=jnp.float32,
        precision=lax.Precision.HIGHEST)


def _mm1(x, w1):
    return pl.pallas_call(
        _mm1_body,
        grid=(NB,),
        in_specs=[pl.BlockSpec((BR, D_IN), lambda i: (i, 0)),
                  pl.BlockSpec((D_IN, D_H), lambda i: (0, 0))],
        out_specs=pl.BlockSpec((BR, D_H), lambda i: (i, 0)),
        out_shape=jax.ShapeDtypeStruct((N, D_H), jnp.float32),
    )(x, w1)


def _scale_body(h_ref, deg_ref, o_ref):
    ns = _norm(deg_ref[0][:, 0:1])
    o_ref[0] = h_ref[:, : D_H // 2] * ns
    o_ref[1] = h_ref[:, D_H // 2:] * ns


def _scale(h1, degc):
    return pl.pallas_call(
        _scale_body,
        grid=(NB,),
        in_specs=[pl.BlockSpec((BR, D_H), lambda i: (i, 0)),
                  pl.BlockSpec((1, BR, 2), lambda i: (i, 0, 0))],
        out_specs=pl.BlockSpec((2, BR, D_H // 2), lambda i: (0, i, 0)),
        out_shape=jax.ShapeDtypeStruct((2, N, D_H // 2), jnp.float32),
    )(h1, degc)


def _mid_body(agg_ref, deg_ref, b1_ref, w2_ref, o_ref):
    a = jnp.concatenate([agg_ref[0], agg_ref[1]], axis=-1)
    ns = _norm(deg_ref[0][:, 0:1])
    nd = _norm(deg_ref[0][:, 1:2])
    z = jnp.maximum(a * nd + b1_ref[...], 0.0) * ns
    y = lax.dot_general(z, w2_ref[...], (((1,), (0,)), ((), ())),
                        preferred_element_type=jnp.float32,
                        precision=lax.Precision.HIGHEST)
    o_ref[...] = y


def _mid(agg1, degc, b1, w2):
    return pl.pallas_call(
        _mid_body,
        grid=(NB,),
        in_specs=[pl.BlockSpec((2, BR, D_H // 2), lambda i: (0, i, 0)),
                  pl.BlockSpec((1, BR, 2), lambda i: (i, 0, 0)),
                  pl.BlockSpec((1, D_H), lambda i: (0, 0)),
                  pl.BlockSpec((D_H, D_OUT), lambda i: (0, 0))],
        out_specs=pl.BlockSpec((BR, D_OUT), lambda i: (i, 0)),
        out_shape=jax.ShapeDtypeStruct((N, D_OUT), jnp.float32),
    )(agg1, degc, b1.reshape(1, D_H), w2)


def _fin_body(agg_ref, deg_ref, b2_ref, o_ref):
    a = agg_ref[0] + agg_ref[1]
    nd = _norm(deg_ref[0][:, 1:2])
    o_ref[...] = a * nd + b2_ref[...]


def _fin(agg2, degc, b2):
    return pl.pallas_call(
        _fin_body,
        grid=(NB,),
        in_specs=[pl.BlockSpec((2, BR, D_OUT), lambda i: (0, i, 0)),
                  pl.BlockSpec((1, BR, 2), lambda i: (i, 0, 0)),
                  pl.BlockSpec((1, D_OUT), lambda i: (0, 0))],
        out_specs=pl.BlockSpec((BR, D_OUT), lambda i: (i, 0)),
        out_shape=jax.ShapeDtypeStruct((N, D_OUT), jnp.float32),
    )(agg2, degc, b2.reshape(1, D_OUT))


def kernel(x, edge_index, W1, b1, W2, b2):
    src = edge_index[0].astype(jnp.int32).reshape(NT, EPT)
    dst = edge_index[1].astype(jnp.int32).reshape(NT, EPT)

    npad1 = EPTP - EPT
    pad_read = (jnp.arange(npad1, dtype=jnp.int32) * 41) % N
    pad_hi = N + (jnp.arange(npad1, dtype=jnp.int32) % (NPAD - N))
    pad_deg = N + (jnp.arange(DEPTP - EPT, dtype=jnp.int32) % (DNPAD - N))

    def _slab(idx, pad, nwin, w):
        npd = nwin * w - idx.shape[1]
        return jnp.concatenate(
            [idx, jnp.broadcast_to(pad[:npd], (NT, npd))], axis=1
        ).reshape(NT, nwin, w)

    src_gather = _slab(src, pad_read, NWIN, W)
    src_slab = jnp.stack([src_gather, src_gather + N])
    dst_slab = _slab(dst, pad_hi, NWIN, W).reshape(NT, NWIN // 2, 2, W)
    deg_slab = jnp.stack([_slab(src, pad_deg, DNWIN, DW),
                          _slab(dst, pad_deg, DNWIN, DW)])

    npad2 = EPTP2 - EPT2
    pad_read2 = (jnp.arange(npad2, dtype=jnp.int32) * 41) % N
    pad_hi2 = N + (jnp.arange(npad2, dtype=jnp.int32) % (NPAD - N))

    def _slab2(idx, pad):
        return jnp.concatenate(
            [idx.reshape(2, NT, EPT2),
             jnp.broadcast_to(pad, (2, NT, npad2))], axis=2
        ).reshape(2, NT, NWIN2, W)

    src2_slab = _slab2(src, pad_read2)
    dst2_slab = _slab2(dst, pad_hi2).reshape(2, NT, NWIN2 // 2, 2, W)

    deg = _deg_kernel(deg_slab)
    degc = deg[:, :N].T.reshape(NB, BR, 2)
    h1 = _mm1(x, W1)
    hcat1 = _scale(h1, degc).reshape(2 * N, D_H // 2)
    agg1 = _agg_l1(hcat1, src_slab, dst_slab)
    h2 = _mid(agg1, degc, b1, W2)
    agg2 = _agg_l2(h2, src2_slab, dst2_slab)
    return _fin(agg2, degc, b2)


# speedup vs baseline: 1.3282x; 1.0282x over previous
import functools

import jax
import jax.numpy as jnp
from jax import lax
from jax.experimental import pallas as pl
from jax.experimental.pallas import tpu as pltpu
from jax.experimental.pallas import tpu_sc as plsc

N = 10000
E = 160000
D_IN = 512
D_H = 256
D_OUT = 128

NT = 16

W = 128
EPT = E // NT
NWIN = 80
EPTP = NWIN * W
NPAD = 10112
RPT = NPAD // NT
RPT_FULL = (RPT // W) * W
RPT_REM = RPT - RPT_FULL

EPT2 = E // (2 * NT)
NWIN2 = 40
EPTP2 = NWIN2 * W

DW = 128
DNWIN = 80
DEPTP = DNWIN * DW
DNPAD = 10240
DRPT = DNPAD // NT

_MESH = plsc.VectorSubcoreMesh(core_axis_name="c", subcore_axis_name="s")


def _zero_fill(buf, rows, cols):
    zeros16 = jnp.zeros((16,), jnp.float32)

    @pl.loop(0, rows)
    def _(i):
        @pl.loop(0, cols, step=16)
        def _(j):
            buf[i, pl.ds(j, 16)] = zeros16


@functools.partial(
    pl.kernel,
    out_type=jax.ShapeDtypeStruct((2, DNPAD), jnp.float32),
    mesh=_MESH,
    scratch_types=[
        pltpu.VMEM((DNWIN, DW), jnp.int32),
        pltpu.VMEM((DW,), jnp.float32),
        pltpu.VMEM((DRPT,), jnp.float32),
        pltpu.VMEM_SHARED((DNPAD,), jnp.float32),
    ],
)
def _deg_kernel(idx_hbm, out_hbm, idx_v, ones_v, zer_v, acc):
    c = lax.axis_index("c")
    s = lax.axis_index("s")
    pltpu.sync_copy(idx_hbm.at[c, s], idx_v)

    ones16 = jnp.ones((16,), jnp.float32)
    zeros16 = jnp.zeros((16,), jnp.float32)

    @pl.loop(0, DW, step=16)
    def _(j):
        ones_v[pl.ds(j, 16)] = ones16

    @pl.loop(0, DRPT, step=16)
    def _(j):
        zer_v[pl.ds(j, 16)] = zeros16

    pltpu.sync_copy(zer_v, acc.at[pl.ds(s * DRPT, DRPT)])
    plsc.subcore_barrier()

    @pl.loop(0, DNWIN)
    def _(w):
        pltpu.sync_copy(ones_v, acc.at[idx_v.at[w]], add=True)

    plsc.subcore_barrier()
    pltpu.sync_copy(acc.at[pl.ds(s * DRPT, DRPT)],
                    out_hbm.at[c, pl.ds(s * DRPT, DRPT)])


def _make_agg_kernel(nwin: int, dst_per_core: bool):
    @functools.partial(
        pl.kernel,
        out_type=jax.ShapeDtypeStruct((2, NPAD, 128), jnp.float32),
        mesh=_MESH,
        scratch_types=[
            pltpu.VMEM((nwin, W), jnp.int32),
            pltpu.VMEM((2, 2, W), jnp.int32),
            pltpu.VMEM((2, W, 128), jnp.float32),
            pltpu.VMEM_SHARED((NPAD, 128), jnp.float32),
            pltpu.SemaphoreType.DMA,
            pltpu.SemaphoreType.DMA,
            pltpu.SemaphoreType.DMA,
            pltpu.SemaphoreType.DMA,
        ],
    )
    def _agg(h_hbm, src_hbm, dst_hbm, out_hbm, src_v, dst_v, buf, acc,
             sem0, sem1, dsem0, dsem1):
        c = lax.axis_index("c")
        s = lax.axis_index("s")
        pltpu.sync_copy(src_hbm.at[c, s], src_v)
        dst_base = dst_hbm.at[c, s] if dst_per_core else dst_hbm.at[s]

        _zero_fill(buf.at[0], W, 128)

        @pl.loop(0, RPT_FULL, step=W)
        def _(r):
            pltpu.sync_copy(buf.at[0], acc.at[pl.ds(s * RPT + r, W)])

        pltpu.sync_copy(buf.at[0].at[pl.ds(0, RPT_REM)],
                        acc.at[pl.ds(s * RPT + RPT_FULL, RPT_REM)])

        plsc.subcore_barrier()

        dsems = (dsem0, dsem1)

        def _wait_g(k, sem):
            pltpu.make_async_copy(h_hbm.at[pl.ds(0, W)], buf.at[k], sem).wait()

        def _gather(w, k, sem):
            pltpu.async_copy(h_hbm.at[src_v.at[w]], buf.at[k], sem)

        def _dst_fetch(pair, k):
            pltpu.async_copy(dst_base.at[pair], dst_v.at[k], dsems[k])

        def _wait_dst(k):
            pltpu.make_async_copy(dst_base.at[0], dst_v.at[k],
                                  dsems[k]).wait()

        def _scatter(bufk, dstk, row):
            pltpu.sync_copy(buf.at[bufk], acc.at[dst_v.at[dstk].at[row]],
                            add=True)

        _dst_fetch(0, 0)
        _dst_fetch(1, 1)
        _gather(0, 0, sem0)

        @pl.loop(0, nwin, step=4)
        def _(w):
            _gather(w + 1, 1, sem1)
            _wait_dst(0)
            _wait_g(0, sem0)
            _scatter(0, 0, 0)

            @pl.when(w + 2 < nwin)
            def _():
                _gather(w + 2, 0, sem0)

            _wait_g(1, sem1)
            _scatter(1, 0, 1)

            @pl.when(w + 4 < nwin)
            def _():
                _dst_fetch(w // 2 + 2, 0)
                _gather(w + 3, 1, sem1)

            @pl.when(w + 4 >= nwin)
            def _():
                _gather(w + 3, 1, sem1)

            _wait_dst(1)
            _wait_g(0, sem0)
            _scatter(0, 1, 0)

            @pl.when(w + 4 < nwin)
            def _():
                _gather(w + 4, 0, sem0)

            _wait_g(1, sem1)
            _scatter(1, 1, 1)

            @pl.when(w + 6 < nwin)
            def _():
                _dst_fetch(w // 2 + 3, 1)

        plsc.subcore_barrier()
        pltpu.sync_copy(acc.at[pl.ds(s * RPT, RPT)],
                        out_hbm.at[c, pl.ds(s * RPT, RPT)])

    return _agg


_agg_l1 = _make_agg_kernel(NWIN, False)
_agg_l2 = _make_agg_kernel(NWIN2, True)


BR = 2000
NB = N // BR


def _norm(deg_row):
    return lax.rsqrt(jnp.where(deg_row > 0.0, deg_row, 1.0))


def _mm1_body(x_ref, w_ref, o_ref):
    o_ref[...] = lax.dot_general(
        x_ref[...], w_ref[...], (((1,), (0,)), ((), ())),
        preferred_element_type=jnp.float32,
        precision=lax.Precision.HIGHEST)


def _mm1(x, w1):
    return pl.pallas_call(
        _mm1_body,
        grid=(NB,),
        in_specs=[pl.BlockSpec((BR, D_IN), lambda i: (i, 0)),
                  pl.BlockSpec((D_IN, D_H), lambda i: (0, 0))],
        out_specs=pl.BlockSpec((BR, D_H), lambda i: (i, 0)),
        out_shape=jax.ShapeDtypeStruct((N, D_H), jnp.float32),
    )(x, w1)


def _scale_body(h_ref, deg_ref, o_ref):
    ns = _norm(deg_ref[0][:, 0:1])
    o_ref[0] = h_ref[:, : D_H // 2] * ns
    o_ref[1] = h_ref[:, D_H // 2:] * ns


def _scale(h1, degc):
    return pl.pallas_call(
        _scale_body,
        grid=(NB,),
        in_specs=[pl.BlockSpec((BR, D_H), lambda i: (i, 0)),
                  pl.BlockSpec((1, BR, 2), lambda i: (i, 0, 0))],
        out_specs=pl.BlockSpec((2, BR, D_H // 2), lambda i: (0, i, 0)),
        out_shape=jax.ShapeDtypeStruct((2, N, D_H // 2), jnp.float32),
    )(h1, degc)


def _mid_body(agg_ref, deg_ref, b1_ref, w2_ref, o_ref):
    a = jnp.concatenate([agg_ref[0], agg_ref[1]], axis=-1)
    ns = _norm(deg_ref[0][:, 0:1])
    nd = _norm(deg_ref[0][:, 1:2])
    z = jnp.maximum(a * nd + b1_ref[...], 0.0) * ns
    y = lax.dot_general(z, w2_ref[...], (((1,), (0,)), ((), ())),
                        preferred_element_type=jnp.float32,
                        precision=lax.Precision.HIGHEST)
    o_ref[...] = y


def _mid(agg1, degc, b1, w2):
    return pl.pallas_call(
        _mid_body,
        grid=(NB,),
        in_specs=[pl.BlockSpec((2, BR, D_H // 2), lambda i: (0, i, 0)),
                  pl.BlockSpec((1, BR, 2), lambda i: (i, 0, 0)),
                  pl.BlockSpec((1, D_H), lambda i: (0, 0)),
                  pl.BlockSpec((D_H, D_OUT), lambda i: (0, 0))],
        out_specs=pl.BlockSpec((BR, D_OUT), lambda i: (i, 0)),
        out_shape=jax.ShapeDtypeStruct((N, D_OUT), jnp.float32),
    )(agg1, degc, b1.reshape(1, D_H), w2)


def _fin_body(agg_ref, deg_ref, b2_ref, o_ref):
    a = agg_ref[0] + agg_ref[1]
    nd = _norm(deg_ref[0][:, 1:2])
    o_ref[...] = a * nd + b2_ref[...]


def _fin(agg2, degc, b2):
    return pl.pallas_call(
        _fin_body,
        grid=(NB,),
        in_specs=[pl.BlockSpec((2, BR, D_OUT), lambda i: (0, i, 0)),
                  pl.BlockSpec((1, BR, 2), lambda i: (i, 0, 0)),
                  pl.BlockSpec((1, D_OUT), lambda i: (0, 0))],
        out_specs=pl.BlockSpec((BR, D_OUT), lambda i: (i, 0)),
        out_shape=jax.ShapeDtypeStruct((N, D_OUT), jnp.float32),
    )(agg2, degc, b2.reshape(1, D_OUT))


def kernel(x, edge_index, W1, b1, W2, b2):
    src = edge_index[0].astype(jnp.int32).reshape(NT, EPT)
    dst = edge_index[1].astype(jnp.int32).reshape(NT, EPT)

    npad1 = EPTP - EPT
    pad_read = (jnp.arange(npad1, dtype=jnp.int32) * 41) % N
    pad_hi = N + (jnp.arange(npad1, dtype=jnp.int32) % (NPAD - N))
    pad_deg = N + (jnp.arange(DEPTP - EPT, dtype=jnp.int32) % (DNPAD - N))

    def _slab(idx, pad, nwin, w):
        npd = nwin * w - idx.shape[1]
        return jnp.concatenate(
            [idx, jnp.broadcast_to(pad[:npd], (NT, npd))], axis=1
        ).reshape(NT, nwin, w)

    src_gather = _slab(src, pad_read, NWIN, W)
    src_slab = jnp.stack([src_gather, src_gather + N])
    dst_slab = _slab(dst, pad_hi, NWIN, W).reshape(NT, NWIN // 2, 2, W)
    deg_slab = jnp.stack([_slab(src, pad_deg, DNWIN, DW),
                          _slab(dst, pad_deg, DNWIN, DW)])

    npad2 = EPTP2 - EPT2
    pad_read2 = (jnp.arange(npad2, dtype=jnp.int32) * 41) % N
    pad_hi2 = N + (jnp.arange(npad2, dtype=jnp.int32) % (NPAD - N))

    def _slab2(idx, pad):
        return jnp.concatenate(
            [idx.reshape(2, NT, EPT2),
             jnp.broadcast_to(pad, (2, NT, npad2))], axis=2
        ).reshape(2, NT, NWIN2, W)

    src2_slab = _slab2(src, pad_read2)
    dst2_slab = _slab2(dst, pad_hi2).reshape(2, NT, NWIN2 // 2, 2, W)

    deg = _deg_kernel(deg_slab)
    degc = deg[:, :N].T.reshape(NB, BR, 2)
    h1 = _mm1(x, W1)
    hcat1 = _scale(h1, degc).reshape(2 * N, D_H // 2)
    agg1 = _agg_l1(hcat1, src_slab, dst_slab)
    h2 = _mid(agg1, degc, b1, W2)
    agg2 = _agg_l2(h2, src2_slab, dst2_slab)
    return _fin(agg2, degc, b2)


# speedup vs baseline: 1.4001x; 1.0541x over previous
import functools

import jax
import jax.numpy as jnp
from jax import lax
from jax.experimental import pallas as pl
from jax.experimental.pallas import tpu as pltpu
from jax.experimental.pallas import tpu_sc as plsc

N = 10000
E = 160000
D_IN = 512
D_H = 256
D_OUT = 128

NT = 16

W = 128
EPT = E // NT
NWIN = 80
EPTP = NWIN * W
NPAD = 10112
RPT = NPAD // NT
RPT_FULL = (RPT // W) * W
RPT_REM = RPT - RPT_FULL

EPT2 = E // (2 * NT)
NWIN2 = 40
EPTP2 = NWIN2 * W

DW = 128
DNWIN = 80
DEPTP = DNWIN * DW
DNPAD = 10240
DRPT = DNPAD // NT

_MESH = plsc.VectorSubcoreMesh(core_axis_name="c", subcore_axis_name="s")


def _zero_fill(buf, rows, cols):
    zeros16 = jnp.zeros((16,), jnp.float32)

    @pl.loop(0, rows)
    def _(i):
        @pl.loop(0, cols, step=16)
        def _(j):
            buf[i, pl.ds(j, 16)] = zeros16


@functools.partial(
    pl.kernel,
    out_type=jax.ShapeDtypeStruct((2, DNPAD), jnp.float32),
    mesh=_MESH,
    scratch_types=[
        pltpu.VMEM((DNWIN, DW), jnp.int32),
        pltpu.VMEM((DW,), jnp.float32),
        pltpu.VMEM((DRPT,), jnp.float32),
        pltpu.VMEM_SHARED((DNPAD,), jnp.float32),
    ],
)
def _deg_kernel(idx_hbm, out_hbm, idx_v, ones_v, zer_v, acc):
    c = lax.axis_index("c")
    s = lax.axis_index("s")
    pltpu.sync_copy(idx_hbm.at[c, s], idx_v)

    ones16 = jnp.ones((16,), jnp.float32)
    zeros16 = jnp.zeros((16,), jnp.float32)

    @pl.loop(0, DW, step=16)
    def _(j):
        ones_v[pl.ds(j, 16)] = ones16

    @pl.loop(0, DRPT, step=16)
    def _(j):
        zer_v[pl.ds(j, 16)] = zeros16

    pltpu.sync_copy(zer_v, acc.at[pl.ds(s * DRPT, DRPT)])
    plsc.subcore_barrier()

    @pl.loop(0, DNWIN)
    def _(w):
        pltpu.sync_copy(ones_v, acc.at[idx_v.at[w]], add=True)

    plsc.subcore_barrier()
    pltpu.sync_copy(acc.at[pl.ds(s * DRPT, DRPT)],
                    out_hbm.at[c, pl.ds(s * DRPT, DRPT)])


def _make_agg_kernel(nwin: int, dst_per_core: bool):
    @functools.partial(
        pl.kernel,
        out_type=jax.ShapeDtypeStruct((2, NPAD, 128), jnp.float32),
        mesh=_MESH,
        scratch_types=[
            pltpu.VMEM((nwin, W), jnp.int32),
            pltpu.VMEM((2, 2, W), jnp.int32),
            pltpu.VMEM((2, W, 128), jnp.float32),
            pltpu.VMEM_SHARED((NPAD, 128), jnp.float32),
            pltpu.SemaphoreType.DMA,
            pltpu.SemaphoreType.DMA,
            pltpu.SemaphoreType.DMA,
            pltpu.SemaphoreType.DMA,
        ],
    )
    def _agg(h_hbm, src_hbm, dst_hbm, out_hbm, src_v, dst_v, buf, acc,
             sem0, sem1, dsem0, dsem1):
        c = lax.axis_index("c")
        s = lax.axis_index("s")
        pltpu.sync_copy(src_hbm.at[c, s], src_v)
        dst_base = dst_hbm.at[c, s] if dst_per_core else dst_hbm.at[s]

        _zero_fill(buf.at[0], W, 128)

        @pl.loop(0, RPT_FULL, step=W)
        def _(r):
            pltpu.sync_copy(buf.at[0], acc.at[pl.ds(s * RPT + r, W)])

        pltpu.sync_copy(buf.at[0].at[pl.ds(0, RPT_REM)],
                        acc.at[pl.ds(s * RPT + RPT_FULL, RPT_REM)])

        plsc.subcore_barrier()

        dsems = (dsem0, dsem1)

        def _wait_g(k, sem):
            pltpu.make_async_copy(h_hbm.at[pl.ds(0, W)], buf.at[k], sem).wait()

        def _gather(w, k, sem):
            pltpu.async_copy(h_hbm.at[src_v.at[w]], buf.at[k], sem)

        def _dst_fetch(pair, k):
            pltpu.async_copy(dst_base.at[pair], dst_v.at[k], dsems[k])

        def _wait_dst(k):
            pltpu.make_async_copy(dst_base.at[0], dst_v.at[k],
                                  dsems[k]).wait()

        def _scatter(bufk, dstk, row):
            pltpu.sync_copy(buf.at[bufk], acc.at[dst_v.at[dstk].at[row]],
                            add=True)

        _dst_fetch(0, 0)
        _dst_fetch(1, 1)
        _gather(0, 0, sem0)

        @pl.loop(0, nwin, step=4)
        def _(w):
            _gather(w + 1, 1, sem1)
            _wait_dst(0)
            _wait_g(0, sem0)
            _scatter(0, 0, 0)

            @pl.when(w + 2 < nwin)
            def _():
                _gather(w + 2, 0, sem0)

            _wait_g(1, sem1)
            _scatter(1, 0, 1)

            @pl.when(w + 4 < nwin)
            def _():
                _dst_fetch(w // 2 + 2, 0)
                _gather(w + 3, 1, sem1)

            @pl.when(w + 4 >= nwin)
            def _():
                _gather(w + 3, 1, sem1)

            _wait_dst(1)
            _wait_g(0, sem0)
            _scatter(0, 1, 0)

            @pl.when(w + 4 < nwin)
            def _():
                _gather(w + 4, 0, sem0)

            _wait_g(1, sem1)
            _scatter(1, 1, 1)

            @pl.when(w + 6 < nwin)
            def _():
                _dst_fetch(w // 2 + 3, 1)

        plsc.subcore_barrier()
        pltpu.sync_copy(acc.at[pl.ds(s * RPT, RPT)],
                        out_hbm.at[c, pl.ds(s * RPT, RPT)])

    return _agg


_agg_l1 = _make_agg_kernel(NWIN, False)
_agg_l2 = _make_agg_kernel(NWIN2, True)


BR = 2000
NB = N // BR


def _norm(deg_row):
    return lax.rsqrt(jnp.where(deg_row > 0.0, deg_row, 1.0))


def _mm1_body(x_ref, w_ref, o_ref):
    o_ref[...] = lax.dot_general(
        x_ref[...], w_ref[...], (((1,), (0,)), ((), ())),
        preferred_element_type=jnp.float32,
        precision=lax.Precision.DEFAULT)


def _mm1(x, w1):
    return pl.pallas_call(
        _mm1_body,
        grid=(NB,),
        in_specs=[pl.BlockSpec((BR, D_IN), lambda i: (i, 0)),
                  pl.BlockSpec((D_IN, D_H), lambda i: (0, 0))],
        out_specs=pl.BlockSpec((BR, D_H), lambda i: (i, 0)),
        out_shape=jax.ShapeDtypeStruct((N, D_H), jnp.float32),
    )(x, w1)


def _scale_body(h_ref, deg_ref, o_ref):
    ns = _norm(deg_ref[0][:, 0:1])
    o_ref[0] = h_ref[:, : D_H // 2] * ns
    o_ref[1] = h_ref[:, D_H // 2:] * ns


def _scale(h1, degc):
    return pl.pallas_call(
        _scale_body,
        grid=(NB,),
        in_specs=[pl.BlockSpec((BR, D_H), lambda i: (i, 0)),
                  pl.BlockSpec((1, BR, 2), lambda i: (i, 0, 0))],
        out_specs=pl.BlockSpec((2, BR, D_H // 2), lambda i: (0, i, 0)),
        out_shape=jax.ShapeDtypeStruct((2, N, D_H // 2), jnp.float32),
    )(h1, degc)


def _mid_body(agg_ref, deg_ref, b1_ref, w2_ref, o_ref):
    a = jnp.concatenate([agg_ref[0], agg_ref[1]], axis=-1)
    ns = _norm(deg_ref[0][:, 0:1])
    nd = _norm(deg_ref[0][:, 1:2])
    z = jnp.maximum(a * nd + b1_ref[...], 0.0) * ns
    y = lax.dot_general(z, w2_ref[...], (((1,), (0,)), ((), ())),
                        preferred_element_type=jnp.float32,
                        precision=lax.Precision.DEFAULT)
    o_ref[...] = y


def _mid(agg1, degc, b1, w2):
    return pl.pallas_call(
        _mid_body,
        grid=(NB,),
        in_specs=[pl.BlockSpec((2, BR, D_H // 2), lambda i: (0, i, 0)),
                  pl.BlockSpec((1, BR, 2), lambda i: (i, 0, 0)),
                  pl.BlockSpec((1, D_H), lambda i: (0, 0)),
                  pl.BlockSpec((D_H, D_OUT), lambda i: (0, 0))],
        out_specs=pl.BlockSpec((BR, D_OUT), lambda i: (i, 0)),
        out_shape=jax.ShapeDtypeStruct((N, D_OUT), jnp.float32),
    )(agg1, degc, b1.reshape(1, D_H), w2)


def _fin_body(agg_ref, deg_ref, b2_ref, o_ref):
    a = agg_ref[0] + agg_ref[1]
    nd = _norm(deg_ref[0][:, 1:2])
    o_ref[...] = a * nd + b2_ref[...]


def _fin(agg2, degc, b2):
    return pl.pallas_call(
        _fin_body,
        grid=(NB,),
        in_specs=[pl.BlockSpec((2, BR, D_OUT), lambda i: (0, i, 0)),
                  pl.BlockSpec((1, BR, 2), lambda i: (i, 0, 0)),
                  pl.BlockSpec((1, D_OUT), lambda i: (0, 0))],
        out_specs=pl.BlockSpec((BR, D_OUT), lambda i: (i, 0)),
        out_shape=jax.ShapeDtypeStruct((N, D_OUT), jnp.float32),
    )(agg2, degc, b2.reshape(1, D_OUT))


def kernel(x, edge_index, W1, b1, W2, b2):
    src = edge_index[0].astype(jnp.int32).reshape(NT, EPT)
    dst = edge_index[1].astype(jnp.int32).reshape(NT, EPT)

    npad1 = EPTP - EPT
    pad_read = (jnp.arange(npad1, dtype=jnp.int32) * 41) % N
    pad_hi = N + (jnp.arange(npad1, dtype=jnp.int32) % (NPAD - N))
    pad_deg = N + (jnp.arange(DEPTP - EPT, dtype=jnp.int32) % (DNPAD - N))

    def _slab(idx, pad, nwin, w):
        npd = nwin * w - idx.shape[1]
        return jnp.concatenate(
            [idx, jnp.broadcast_to(pad[:npd], (NT, npd))], axis=1
        ).reshape(NT, nwin, w)

    src_gather = _slab(src, pad_read, NWIN, W)
    src_slab = jnp.stack([src_gather, src_gather + N])
    dst_slab = _slab(dst, pad_hi, NWIN, W).reshape(NT, NWIN // 2, 2, W)
    deg_slab = jnp.stack([_slab(src, pad_deg, DNWIN, DW),
                          _slab(dst, pad_deg, DNWIN, DW)])

    npad2 = EPTP2 - EPT2
    pad_read2 = (jnp.arange(npad2, dtype=jnp.int32) * 41) % N
    pad_hi2 = N + (jnp.arange(npad2, dtype=jnp.int32) % (NPAD - N))

    def _slab2(idx, pad):
        return jnp.concatenate(
            [idx.reshape(2, NT, EPT2),
             jnp.broadcast_to(pad, (2, NT, npad2))], axis=2
        ).reshape(2, NT, NWIN2, W)

    src2_slab = _slab2(src, pad_read2)
    dst2_slab = _slab2(dst, pad_hi2).reshape(2, NT, NWIN2 // 2, 2, W)

    deg = _deg_kernel(deg_slab)
    degc = deg[:, :N].T.reshape(NB, BR, 2)
    h1 = _mm1(x, W1)
    hcat1 = _scale(h1, degc).reshape(2 * N, D_H // 2)
    agg1 = _agg_l1(hcat1, src_slab, dst_slab)
    h2 = _mid(agg1, degc, b1, W2)
    agg2 = _agg_l2(h2, src2_slab, dst2_slab)
    return _fin(agg2, degc, b2)
